# baseline (speedup 1.0000x reference)
import functools

import jax
import jax.numpy as jnp
from jax import lax
from jax.experimental import pallas as pl
from jax.experimental.pallas import tpu as pltpu
from jax.experimental.pallas import tpu_sc as plsc

NC = 2
NS = 16
NW = NC * NS
L = 16

T0 = 128
TS = 64
CHUNK = 64
RU = 4
HW = 80
BR = 8192


def _tc_body(w_ref, cw_ref, m_ref, off_ref):
    wp = w_ref[...] + jnp.float32(0.01)
    rows = lax.broadcasted_iota(jnp.int32, (T0, T0), 0)
    cols = lax.broadcasted_iota(jnp.int32, (T0, T0), 1)
    triu = (rows <= cols).astype(jnp.bfloat16)
    hi = wp.astype(jnp.bfloat16)
    lo = (wp - hi.astype(jnp.float32)).astype(jnp.bfloat16)
    cw = (jnp.dot(hi, triu, preferred_element_type=jnp.float32)
          + jnp.dot(lo, triu, preferred_element_type=jnp.float32))
    cw_ref[...] = cw
    s = cw[:, T0 - 1:T0]
    t = cw * (jnp.float32(TS) / s) - jnp.float32(0.5)
    m = jnp.clip(jnp.ceil(t).astype(jnp.int32), 0, TS)
    m_ref[...] = m
    ones_col = jnp.ones((T0, 1), jnp.float32)
    offs = [jnp.zeros((m.shape[0], 1), jnp.int32)]
    for b in range(1, TS // L):
        mask = (m <= L * b - 1).astype(jnp.float32)
        cnt = jnp.dot(mask, ones_col, preferred_element_type=jnp.float32)
        offs.append(cnt.astype(jnp.int32))
    offs.append(jnp.zeros((m.shape[0], L - TS // L), jnp.int32))
    off_ref[...] = jnp.concatenate(offs, axis=-1)


def _tc_stage(weights, q, nq):
    n = weights.shape[0]
    h = n // nq
    blk0 = q * (h // BR)
    return pl.pallas_call(
        _tc_body,
        grid=(h // BR,),
        in_specs=[pl.BlockSpec((BR, T0), lambda i: (i + blk0, 0))],
        out_specs=[
            pl.BlockSpec((BR, T0), lambda i: (i, 0)),
            pl.BlockSpec((BR, T0), lambda i: (i, 0)),
            pl.BlockSpec((BR, L), lambda i: (i, 0)),
        ],
        out_shape=[
            jax.ShapeDtypeStruct((h, T0), jnp.float32),
            jax.ShapeDtypeStruct((h, T0), jnp.int32),
            jax.ShapeDtypeStruct((h, L), jnp.int32),
        ],
    )(weights)


def _process_row(r, bins_v, cw_v, m_v, off_v, out_v, u_vecs, h_v, ones16):
    row_idx = jnp.full((L,), r, jnp.int32)
    h_base = (r & 15) * HW

    s_vec = plsc.load_gather(cw_v, [row_idx, jnp.full((L,), T0 - 1,
                                                      jnp.int32)])

    zero16 = jnp.zeros((L,), jnp.int32)
    for i in range(4):
        h_v[pl.ds(h_base + L * i, L)] = zero16

    for i in range(T0 // L):
        mv = m_v[r, pl.ds(L * i, L)]
        plsc.addupdate_scatter(h_v, [mv + h_base], ones16)

    for b in range(TS // L):
        hv = h_v[pl.ds(h_base + L * b, L)]
        c = plsc.cumsum(hv)
        if b > 0:
            c = c + plsc.load_gather(
                off_v, [row_idx, jnp.full((L,), b, jnp.int32)])
        v = u_vecs[b] * s_vec
        bins_g0 = plsc.load_gather(bins_v, [row_idx, c])
        bins_g1 = plsc.load_gather(bins_v, [row_idx, jnp.minimum(c + 1, T0)])
        cg0 = plsc.load_gather(cw_v, [row_idx, jnp.maximum(c - 1, 0)])
        cdf_g0 = jnp.where(c > 0, cg0, jnp.float32(0.0))
        cdf_g1 = plsc.load_gather(cw_v, [row_idx, jnp.minimum(c, T0 - 1)])
        denom = cdf_g1 - cdf_g0
        pos = denom > 0
        t = jnp.where(
            pos, (v - cdf_g0) / jnp.where(pos, denom, jnp.float32(1.0)),
            jnp.float32(0.0))
        t = jnp.clip(t, 0.0, 1.0)
        out_v[r, pl.ds(L * b, L)] = bins_g0 + t * (bins_g1 - bins_g0)


def _sc_body(bins_hbm, cw_hbm, m_hbm, off_hbm, u_hbm, out_hbm,
             bins_v, cw_v, m_v, off_v, out_v, u_v, h_v,
             sbi, scw, smi, sof, sout, *, rows0):
    h = cw_hbm.shape[0]
    rows_per_w = h // NW
    n_chunks = rows_per_w // CHUNK
    wid = lax.axis_index("s") * NC + lax.axis_index("c")
    base = wid * rows_per_w
    base_f = rows0 + base

    pltpu.sync_copy(u_hbm, u_v)
    ones16 = jnp.ones((L,), jnp.int32)
    u_vecs = [u_v[pl.ds(L * b, L)] for b in range(TS // L)]

    srcs = (bins_hbm, cw_hbm, m_hbm, off_hbm)
    dsts = (bins_v, cw_v, m_v, off_v)
    sems = (sbi, scw, smi, sof)
    bases = (base_f, base, base, base)

    def start_in(ci, buf):
        for src, dst, sem, b0 in zip(srcs, dsts, sems, bases):
            pltpu.async_copy(src.at[pl.ds(b0 + ci * CHUNK, CHUNK)],
                             dst.at[buf], sem[buf])

    def wait_in(buf):
        for src, dst, sem in zip(srcs, dsts, sems):
            pltpu.make_async_copy(src.at[pl.ds(0, CHUNK)], dst.at[buf],
                                  sem[buf]).wait()

    def wait_out(buf):
        pltpu.make_async_copy(out_v.at[buf], out_hbm.at[pl.ds(0, CHUNK)],
                              sout[buf]).wait()

    def process(ci, buf):
        @plsc.parallel_loop(0, CHUNK, 1, unroll=RU)
        def _rows(r):
            _process_row(r, bins_v.at[buf], cw_v.at[buf],
                         m_v.at[buf], off_v.at[buf], out_v.at[buf],
                         u_vecs, h_v, ones16)
        pltpu.async_copy(out_v.at[buf],
                         out_hbm.at[pl.ds(base + ci * CHUNK, CHUNK)],
                         sout[buf])

    start_in(0, 0)

    def chunk_pair(hh, _):
        ci0 = 2 * hh

        @pl.when(hh > 0)
        def _w0():
            wait_out(0)
        start_in(ci0 + 1, 1)
        wait_in(0)
        process(ci0, 0)

        @pl.when(hh > 0)
        def _w1():
            wait_out(1)

        @pl.when(hh < (n_chunks // 2) - 1)
        def _pf():
            start_in(ci0 + 2, 0)
        wait_in(1)
        process(ci0 + 1, 1)
        return _

    lax.fori_loop(0, n_chunks // 2, chunk_pair, None)
    wait_out(0)
    wait_out(1)


def _sc_stage(bins, cw, m, off, u, q, nq):
    h = bins.shape[0] // nq
    mesh = plsc.VectorSubcoreMesh(
        core_axis_name="c", subcore_axis_name="s", num_cores=NC,
        num_subcores=NS)
    f = pl.kernel(
        functools.partial(_sc_body, rows0=q * h),
        out_type=jax.ShapeDtypeStruct((h, TS), jnp.float32),
        mesh=mesh,
        scratch_types=[
            pltpu.VMEM((2, CHUNK, T0 + 1), jnp.float32),
            pltpu.VMEM((2, CHUNK, T0), jnp.float32),
            pltpu.VMEM((2, CHUNK, T0), jnp.int32),
            pltpu.VMEM((2, CHUNK, L), jnp.int32),
            pltpu.VMEM((2, CHUNK, TS), jnp.float32),
            pltpu.VMEM((TS,), jnp.float32),
            pltpu.VMEM((16 * HW,), jnp.int32),
            [pltpu.SemaphoreType.DMA] * 2,
            [pltpu.SemaphoreType.DMA] * 2,
            [pltpu.SemaphoreType.DMA] * 2,
            [pltpu.SemaphoreType.DMA] * 2,
            [pltpu.SemaphoreType.DMA] * 2,
        ],
        compiler_params=pltpu.CompilerParams(needs_layout_passes=False),
    )
    return f(bins, cw, m, off, u)


NQ = 2


def kernel(bins, weights, T):
    tf = jnp.asarray(T, jnp.float32)
    u = (0.5 / tf + jnp.arange(TS, dtype=jnp.float32) * ((1.0 - 1.0 / tf)
                                                         / (TS - 1)))
    u = u.astype(jnp.float32)
    outs = []
    for q in range(NQ):
        cw, m, off = _tc_stage(weights, q, NQ)
        outs.append(_sc_stage(bins, cw, m, off, u, q, NQ))
    return outs[0] if NQ == 1 else jnp.concatenate(outs, 0)


# speedup vs baseline: 1.0629x; 1.0629x over previous
import functools

import jax
import jax.numpy as jnp
from jax import lax
from jax.experimental import pallas as pl
from jax.experimental.pallas import tpu as pltpu
from jax.experimental.pallas import tpu_sc as plsc

NC = 2
NS = 16
NW = NC * NS
L = 16

T0 = 128
TS = 64
CHUNK = 64
RU = 4
HW = 80
BR = 8192


def _tc_body(w_ref, cw_ref, m_ref, off_ref):
    wp = w_ref[...] + jnp.float32(0.01)
    rows = lax.broadcasted_iota(jnp.int32, (T0, T0), 0)
    cols = lax.broadcasted_iota(jnp.int32, (T0, T0), 1)
    triu = (rows <= cols).astype(jnp.bfloat16)
    hi = wp.astype(jnp.bfloat16)
    lo = (wp - hi.astype(jnp.float32)).astype(jnp.bfloat16)
    cw = (jnp.dot(hi, triu, preferred_element_type=jnp.float32)
          + jnp.dot(lo, triu, preferred_element_type=jnp.float32))
    cw_ref[...] = cw
    s = cw[:, T0 - 1:T0]
    t = cw * (jnp.float32(TS) / s) - jnp.float32(0.5)
    m = jnp.clip(jnp.ceil(t).astype(jnp.int32), 0, TS)
    m_ref[...] = m
    ones_col = jnp.ones((T0, 1), jnp.float32)
    offs = [jnp.zeros((m.shape[0], 1), jnp.int32)]
    for b in range(1, TS // L):
        mask = (m <= L * b - 1).astype(jnp.float32)
        cnt = jnp.dot(mask, ones_col, preferred_element_type=jnp.float32)
        offs.append(cnt.astype(jnp.int32))
    offs.append(jnp.zeros((m.shape[0], L - TS // L), jnp.int32))
    off_ref[...] = jnp.concatenate(offs, axis=-1)


def _tc_stage(weights, q, nq):
    n = weights.shape[0]
    h = n // nq
    blk0 = q * (h // BR)
    return pl.pallas_call(
        _tc_body,
        grid=(h // BR,),
        in_specs=[pl.BlockSpec((BR, T0), lambda i: (i + blk0, 0))],
        out_specs=[
            pl.BlockSpec((BR, T0), lambda i: (i, 0)),
            pl.BlockSpec((BR, T0), lambda i: (i, 0)),
            pl.BlockSpec((BR, L), lambda i: (i, 0)),
        ],
        out_shape=[
            jax.ShapeDtypeStruct((h, T0), jnp.float32),
            jax.ShapeDtypeStruct((h, T0), jnp.int32),
            jax.ShapeDtypeStruct((h, L), jnp.int32),
        ],
    )(weights)


def _process_row(r, bins_v, cw_v, m_v, off_v, out_v, u_vecs, h_v, ones16):
    row_idx = jnp.full((L,), r, jnp.int32)
    h_base = (r & 15) * HW

    s_vec = plsc.load_gather(cw_v, [row_idx, jnp.full((L,), T0 - 1,
                                                      jnp.int32)])

    zero16 = jnp.zeros((L,), jnp.int32)
    for i in range(4):
        h_v[pl.ds(h_base + L * i, L)] = zero16

    for i in range(T0 // L):
        mv = m_v[r, pl.ds(L * i, L)]
        plsc.addupdate_scatter(h_v, [mv + h_base], ones16)

    for b in range(TS // L):
        hv = h_v[pl.ds(h_base + L * b, L)]
        c = plsc.cumsum(hv)
        if b > 0:
            c = c + plsc.load_gather(
                off_v, [row_idx, jnp.full((L,), b, jnp.int32)])
        v = u_vecs[b] * s_vec
        bins_g0 = plsc.load_gather(bins_v, [row_idx, c])
        bins_g1 = plsc.load_gather(bins_v, [row_idx, jnp.minimum(c + 1, T0)])
        cg0 = plsc.load_gather(cw_v, [row_idx, jnp.maximum(c - 1, 0)])
        cdf_g0 = jnp.where(c > 0, cg0, jnp.float32(0.0))
        cdf_g1 = plsc.load_gather(cw_v, [row_idx, jnp.minimum(c, T0 - 1)])
        denom = cdf_g1 - cdf_g0
        pos = denom > 0
        t = jnp.where(
            pos, (v - cdf_g0) / jnp.where(pos, denom, jnp.float32(1.0)),
            jnp.float32(0.0))
        t = jnp.clip(t, 0.0, 1.0)
        out_v[r, pl.ds(L * b, L)] = bins_g0 + t * (bins_g1 - bins_g0)


def _sc_body(bins_hbm, cw_hbm, m_hbm, off_hbm, u_hbm, out_hbm,
             bins_v, cw_v, m_v, off_v, out_v, u_v, h_v,
             sbi, scw, smi, sof, sout, *, rows0):
    h = cw_hbm.shape[0]
    rows_per_w = h // NW
    n_chunks = rows_per_w // CHUNK
    wid = lax.axis_index("s") * NC + lax.axis_index("c")
    base = wid * rows_per_w
    base_f = rows0 + base

    pltpu.sync_copy(u_hbm, u_v)
    ones16 = jnp.ones((L,), jnp.int32)
    u_vecs = [u_v[pl.ds(L * b, L)] for b in range(TS // L)]

    srcs = (bins_hbm, cw_hbm, m_hbm, off_hbm)
    dsts = (bins_v, cw_v, m_v, off_v)
    sems = (sbi, scw, smi, sof)
    bases = (base_f, base, base, base)

    def start_in(ci, buf):
        for src, dst, sem, b0 in zip(srcs, dsts, sems, bases):
            pltpu.async_copy(src.at[pl.ds(b0 + ci * CHUNK, CHUNK)],
                             dst.at[buf], sem[buf])

    def wait_in(buf):
        for src, dst, sem in zip(srcs, dsts, sems):
            pltpu.make_async_copy(src.at[pl.ds(0, CHUNK)], dst.at[buf],
                                  sem[buf]).wait()

    def wait_out(buf):
        pltpu.make_async_copy(out_v.at[buf], out_hbm.at[pl.ds(0, CHUNK)],
                              sout[buf]).wait()

    def process(ci, buf):
        @plsc.parallel_loop(0, CHUNK, 1, unroll=RU)
        def _rows(r):
            _process_row(r, bins_v.at[buf], cw_v.at[buf],
                         m_v.at[buf], off_v.at[buf], out_v.at[buf],
                         u_vecs, h_v, ones16)
        pltpu.async_copy(out_v.at[buf],
                         out_hbm.at[pl.ds(base + ci * CHUNK, CHUNK)],
                         sout[buf])

    start_in(0, 0)

    def chunk_pair(hh, _):
        ci0 = 2 * hh

        @pl.when(hh > 0)
        def _w0():
            wait_out(0)
        start_in(ci0 + 1, 1)
        wait_in(0)
        process(ci0, 0)

        @pl.when(hh > 0)
        def _w1():
            wait_out(1)

        @pl.when(hh < (n_chunks // 2) - 1)
        def _pf():
            start_in(ci0 + 2, 0)
        wait_in(1)
        process(ci0 + 1, 1)
        return _

    lax.fori_loop(0, n_chunks // 2, chunk_pair, None)
    wait_out(0)
    wait_out(1)


def _sc_stage(bins, cw, m, off, u, q, nq):
    h = bins.shape[0] // nq
    mesh = plsc.VectorSubcoreMesh(
        core_axis_name="c", subcore_axis_name="s", num_cores=NC,
        num_subcores=NS)
    f = pl.kernel(
        functools.partial(_sc_body, rows0=q * h),
        out_type=jax.ShapeDtypeStruct((h, TS), jnp.float32),
        mesh=mesh,
        scratch_types=[
            pltpu.VMEM((2, CHUNK, T0 + 1), jnp.float32),
            pltpu.VMEM((2, CHUNK, T0), jnp.float32),
            pltpu.VMEM((2, CHUNK, T0), jnp.int32),
            pltpu.VMEM((2, CHUNK, L), jnp.int32),
            pltpu.VMEM((2, CHUNK, TS), jnp.float32),
            pltpu.VMEM((TS,), jnp.float32),
            pltpu.VMEM((16 * HW,), jnp.int32),
            [pltpu.SemaphoreType.DMA] * 2,
            [pltpu.SemaphoreType.DMA] * 2,
            [pltpu.SemaphoreType.DMA] * 2,
            [pltpu.SemaphoreType.DMA] * 2,
            [pltpu.SemaphoreType.DMA] * 2,
        ],
        compiler_params=pltpu.CompilerParams(needs_layout_passes=False),
    )
    return f(bins, cw, m, off, u)


NQ = 1


def kernel(bins, weights, T):
    tf = jnp.asarray(T, jnp.float32)
    u = (0.5 / tf + jnp.arange(TS, dtype=jnp.float32) * ((1.0 - 1.0 / tf)
                                                         / (TS - 1)))
    u = u.astype(jnp.float32)
    outs = []
    for q in range(NQ):
        cw, m, off = _tc_stage(weights, q, NQ)
        outs.append(_sc_stage(bins, cw, m, off, u, q, NQ))
    return outs[0] if NQ == 1 else jnp.concatenate(outs, 0)
